# Initial kernel scaffold; baseline (speedup 1.0000x reference)
import jax
import jax.numpy as jnp
from jax.experimental import pallas as pl


def kernel(x, Wg, bg, We, be):
    raise NotImplementedError("write your pallas kernel here")


# speedup vs baseline: 1.8273x; 1.8273x over previous
import functools

import jax
import jax.numpy as jnp
from jax.experimental import pallas as pl
from jax.experimental.pallas import tpu as pltpu


def _gate_body(x_ref, wg_ref, bg_ref, sel_ref):
    logits = (
        jax.lax.dot_general(
            x_ref[...], wg_ref[...], (((1,), (0,)), ((), ())),
            preferred_element_type=jnp.float32,
            precision=jax.lax.Precision.DEFAULT,
        )
        + bg_ref[...]
    )
    tm, e = logits.shape
    ecol = jax.lax.broadcasted_iota(jnp.int32, (tm, e), 1)
    m1 = jnp.max(logits, axis=1, keepdims=True)
    idx1 = jnp.min(jnp.where(logits == m1, ecol, e), axis=1, keepdims=True)
    mask1 = ecol == idx1
    l2 = jnp.where(mask1, -jnp.inf, logits)
    m2 = jnp.max(l2, axis=1, keepdims=True)
    idx2 = jnp.min(jnp.where(l2 == m2, ecol, e), axis=1, keepdims=True)
    mask2 = ecol == idx2
    sel_ref[...] = (mask1.astype(jnp.float32) + mask2.astype(jnp.float32)) * 0.5


def _expert_body(x_ref, we_ref, be_ref, sel_ref, o_ref):
    e = pl.program_id(1)

    @pl.when(e == 0)
    def _init():
        o_ref[...] = jnp.zeros_like(o_ref)

    y = jax.lax.dot_general(
        x_ref[...], we_ref[0], (((1,), (0,)), ((), ())),
        preferred_element_type=jnp.float32,
        precision=jax.lax.Precision.HIGHEST,
    )
    y = jnp.maximum(y + be_ref[0], 0.0)
    sel = sel_ref[...]
    ecol = jax.lax.broadcasted_iota(jnp.int32, sel.shape, 1)
    sel_col = jnp.sum(jnp.where(ecol == e, sel, 0.0), axis=1, keepdims=True)
    o_ref[...] += sel_col * y


@functools.partial(jax.jit, static_argnums=())
def kernel(x, Wg, bg, We, be):
    n, s, v = x.shape
    e = Wg.shape[1]
    out = We.shape[2]
    t = n * s
    xt = x.reshape(t, v)
    tm = 1024
    nt = t // tm

    sel = pl.pallas_call(
        _gate_body,
        grid=(nt,),
        in_specs=[
            pl.BlockSpec((tm, v), lambda i: (i, 0)),
            pl.BlockSpec((v, e), lambda i: (0, 0)),
            pl.BlockSpec((1, e), lambda i: (0, 0)),
        ],
        out_specs=pl.BlockSpec((tm, e), lambda i: (i, 0)),
        out_shape=jax.ShapeDtypeStruct((t, e), jnp.float32),
    )(xt, Wg, bg.reshape(1, e))

    o = pl.pallas_call(
        _expert_body,
        grid=(nt, e),
        in_specs=[
            pl.BlockSpec((tm, v), lambda i, j: (i, 0)),
            pl.BlockSpec((1, v, out), lambda i, j: (j, 0, 0)),
            pl.BlockSpec((1, 1, out), lambda i, j: (j, 0, 0)),
            pl.BlockSpec((tm, e), lambda i, j: (i, 0)),
        ],
        out_specs=pl.BlockSpec((tm, out), lambda i, j: (i, 0)),
        out_shape=jax.ShapeDtypeStruct((t, out), jnp.float32),
    )(xt, We, be.reshape(e, 1, out), sel)

    return o.reshape(n, s, out)


# speedup vs baseline: 8.0322x; 4.3958x over previous
import functools

import jax
import jax.numpy as jnp
from jax.experimental import pallas as pl
from jax.experimental.pallas import tpu as pltpu


def _gate_body(x_ref, wg_ref, bg_ref, sel_ref):
    logits = (
        jax.lax.dot_general(
            x_ref[...], wg_ref[...], (((1,), (0,)), ((), ())),
            preferred_element_type=jnp.float32,
            precision=jax.lax.Precision.DEFAULT,
        )
        + bg_ref[...]
    )
    tm, e = logits.shape
    ecol = jax.lax.broadcasted_iota(jnp.int32, (tm, e), 1)
    m1 = jnp.max(logits, axis=1, keepdims=True)
    idx1 = jnp.min(jnp.where(logits == m1, ecol, e), axis=1, keepdims=True)
    mask1 = ecol == idx1
    l2 = jnp.where(mask1, -jnp.inf, logits)
    m2 = jnp.max(l2, axis=1, keepdims=True)
    idx2 = jnp.min(jnp.where(l2 == m2, ecol, e), axis=1, keepdims=True)
    mask2 = ecol == idx2
    sel_ref[...] = (mask1.astype(jnp.float32) + mask2.astype(jnp.float32)) * 0.5


def _expert_body(x_ref, we_ref, be_ref, sel_ref, o_ref):
    e = pl.program_id(1)

    @pl.when(e == 0)
    def _init():
        o_ref[...] = jnp.zeros_like(o_ref)

    y = jax.lax.dot_general(
        x_ref[...], we_ref[0], (((1,), (0,)), ((), ())),
        preferred_element_type=jnp.float32,
        precision=jax.lax.Precision.DEFAULT,
    )
    y = jnp.maximum(y + be_ref[0], 0.0)
    sel = sel_ref[...]
    ecol = jax.lax.broadcasted_iota(jnp.int32, sel.shape, 1)
    sel_col = jnp.sum(jnp.where(ecol == e, sel, 0.0), axis=1, keepdims=True)
    o_ref[...] += sel_col * y


@functools.partial(jax.jit, static_argnums=())
def kernel(x, Wg, bg, We, be):
    n, s, v = x.shape
    e = Wg.shape[1]
    out = We.shape[2]
    t = n * s
    xt = x.reshape(t, v)
    tm = 1024
    nt = t // tm

    sel = pl.pallas_call(
        _gate_body,
        grid=(nt,),
        in_specs=[
            pl.BlockSpec((tm, v), lambda i: (i, 0)),
            pl.BlockSpec((v, e), lambda i: (0, 0)),
            pl.BlockSpec((1, e), lambda i: (0, 0)),
        ],
        out_specs=pl.BlockSpec((tm, e), lambda i: (i, 0)),
        out_shape=jax.ShapeDtypeStruct((t, e), jnp.float32),
    )(xt, Wg, bg.reshape(1, e))

    o = pl.pallas_call(
        _expert_body,
        grid=(nt, e),
        in_specs=[
            pl.BlockSpec((tm, v), lambda i, j: (i, 0)),
            pl.BlockSpec((1, v, out), lambda i, j: (j, 0, 0)),
            pl.BlockSpec((1, 1, out), lambda i, j: (j, 0, 0)),
            pl.BlockSpec((tm, e), lambda i, j: (i, 0)),
        ],
        out_specs=pl.BlockSpec((tm, out), lambda i, j: (i, 0)),
        out_shape=jax.ShapeDtypeStruct((t, out), jnp.float32),
    )(xt, We, be.reshape(e, 1, out), sel)

    return o.reshape(n, s, out)
